# baseline (speedup 1.0000x reference)
import functools

import jax
import jax.numpy as jnp
from jax import lax
from jax.experimental import pallas as pl
from jax.experimental.pallas import tpu as pltpu
from jax.experimental.pallas import tpu_sc as plsc

B, S, D = 1, 2048, 768
E, K = 16, 2
DFF = 2 * D
MLPH = 4 * D
NUM_STEPS = 2
N = B * S
NK = N * K

BLK = 128
NB = NK // BLK + E
P = NB * BLK

SC_NC, SC_NS = 2, 16
NW = SC_NC * SC_NS

_EPS = 1e-6
_NEG = -1e30


def _router_body(x_ref, wr_ref, logits_ref, i1_ref, i2_ref, w1_ref, w2_ref,
                 lb_ref):
    x = x_ref[...]
    logits = lax.dot_general(x, wr_ref[...], (((1,), (1,)), ((), ())),
                             preferred_element_type=jnp.float32)
    logits_ref[...] = logits
    col = lax.broadcasted_iota(jnp.int32, logits.shape, 1)
    valid = col < E
    lm = jnp.where(valid, logits, _NEG)
    m1 = jnp.max(lm, axis=1, keepdims=True)
    i1 = jnp.min(jnp.where(lm == m1, col, 127), axis=1, keepdims=True)
    lm2 = jnp.where(col == i1, _NEG, lm)
    m2 = jnp.max(lm2, axis=1, keepdims=True)
    i2 = jnp.min(jnp.where(lm2 == m2, col, 127), axis=1, keepdims=True)
    ex = jnp.where(valid, jnp.exp(lm - m1), 0.0)
    ssum = jnp.sum(ex, axis=1, keepdims=True)
    probs = ex / ssum
    p1 = 1.0 / ssum
    p2 = jnp.exp(m2 - m1) / ssum
    tot = p1 + p2
    i1_ref[...] = i1
    i2_ref[...] = i2
    w1_ref[...] = p1 / tot
    w2_ref[...] = p2 / tot
    assign = (col == i1).astype(jnp.float32) + (col == i2).astype(jnp.float32)
    ca = jnp.sum(assign, axis=0)
    cp = jnp.sum(probs, axis=0)
    lb_ref[...] = ((float(E) / (K * N * N)) * jnp.sum(ca * cp)).reshape(1, 1)


def _router(x_flat, wr_pad):
    return pl.pallas_call(
        _router_body,
        out_shape=(
            jax.ShapeDtypeStruct((N, 128), jnp.float32),
            jax.ShapeDtypeStruct((N, 1), jnp.int32),
            jax.ShapeDtypeStruct((N, 1), jnp.int32),
            jax.ShapeDtypeStruct((N, 1), jnp.float32),
            jax.ShapeDtypeStruct((N, 1), jnp.float32),
            jax.ShapeDtypeStruct((1, 1), jnp.float32),
        ),
    )(x_flat, wr_pad)


@functools.lru_cache(maxsize=None)
def _make_sc_gather(n_out, d):
    rpw = n_out // NW
    ch = min(rpw, 64)
    n_ch = rpw // ch
    mesh = plsc.VectorSubcoreMesh(core_axis_name="c", subcore_axis_name="s",
                                  num_cores=SC_NC, num_subcores=SC_NS)

    @functools.partial(
        pl.kernel,
        out_type=jax.ShapeDtypeStruct((n_out, d), jnp.float32),
        mesh=mesh,
        scratch_types=[
            pltpu.VMEM((ch,), jnp.int32),
            pltpu.VMEM((ch,), jnp.int32),
            pltpu.VMEM((2, ch, d), jnp.float32),
            pltpu.SemaphoreType.DMA,
            pltpu.SemaphoreType.DMA,
            pltpu.SemaphoreType.DMA,
            pltpu.SemaphoreType.DMA,
            pltpu.SemaphoreType.DMA,
            pltpu.SemaphoreType.DMA,
        ],
    )
    def gather_k(table_hbm, idx_hbm, out_hbm, ix0, ix1, rows_v,
                 g0, g1, w0, w1, i0, i1):
        wid = lax.axis_index("s") * SC_NC + lax.axis_index("c")
        base = wid * rpw
        ibuf = (ix0, ix1)
        gsem = (g0, g1)
        wsem = (w0, w1)
        isem = (i0, i1)
        gathers = [None] * n_ch
        writes = [None] * n_ch
        iloads = [None] * n_ch
        iloads[0] = pltpu.async_copy(idx_hbm.at[pl.ds(base, ch)], ix0, i0)
        for c in range(n_ch):
            s = c % 2
            if c >= 2:
                writes[c - 2].wait()
            iloads[c].wait()
            gathers[c] = pltpu.async_copy(
                table_hbm.at[ibuf[s]], rows_v.at[s], gsem[s])
            if c >= 1:
                gathers[c - 1].wait()
                writes[c - 1] = pltpu.async_copy(
                    rows_v.at[(c - 1) % 2],
                    out_hbm.at[pl.ds(base + (c - 1) * ch, ch)],
                    wsem[(c - 1) % 2])
            if c + 1 < n_ch:
                iloads[c + 1] = pltpu.async_copy(
                    idx_hbm.at[pl.ds(base + (c + 1) * ch, ch)],
                    ibuf[(c + 1) % 2], isem[(c + 1) % 2])
        gathers[n_ch - 1].wait()
        writes[n_ch - 1] = pltpu.async_copy(
            rows_v.at[(n_ch - 1) % 2],
            out_hbm.at[pl.ds(base + (n_ch - 1) * ch, ch)],
            wsem[(n_ch - 1) % 2])
        if n_ch >= 2:
            writes[n_ch - 2].wait()
        writes[n_ch - 1].wait()

    return gather_k


def _sc_gather_P(table, idx):
    return _make_sc_gather(P, D)(table, idx)


def _sc_gather_N(table, idx):
    return _make_sc_gather(N, D)(table, idx)


def _ffn_body(be_ref, xg_ref, w1_ref, w2_ref, ws_ref, y_ref):
    xb = xg_ref[...]
    h = lax.dot_general(xb, w1_ref[0], (((1,), (1,)), ((), ())),
                        preferred_element_type=jnp.float32)
    h = 0.5 * h * (1.0 + lax.erf(h * (2.0 ** -0.5)))
    y = lax.dot_general(h, w2_ref[0], (((1,), (1,)), ((), ())),
                        preferred_element_type=jnp.float32)
    y_ref[...] = y * ws_ref[...]


def _ffn(block_expert, xg, W1, W2, ws):
    grid_spec = pltpu.PrefetchScalarGridSpec(
        num_scalar_prefetch=1,
        grid=(NB,),
        in_specs=[
            pl.BlockSpec((BLK, D), lambda b, be: (b, 0)),
            pl.BlockSpec((1, DFF, D), lambda b, be: (be[b], 0, 0)),
            pl.BlockSpec((1, D, DFF), lambda b, be: (be[b], 0, 0)),
            pl.BlockSpec((BLK, 1), lambda b, be: (b, 0)),
        ],
        out_specs=pl.BlockSpec((BLK, D), lambda b, be: (b, 0)),
    )
    return pl.pallas_call(
        _ffn_body,
        grid_spec=grid_spec,
        out_shape=jax.ShapeDtypeStruct((P, D), jnp.float32),
    )(block_expert, xg, W1, W2, ws)


def _rms(v, w):
    return v * lax.rsqrt(jnp.mean(v * v, axis=-1, keepdims=True) + _EPS) * w


def _recur_body(eo0_ref, eo1_ref, st_ref, a_s_ref, a_x_ref, n1_ref, n2_ref,
                n3_ref, n4_ref, wv_ref, wo_ref, m1_ref, m2_ref, out_ref):
    static_input = eo0_ref[...] + eo1_ref[...]
    state = st_ref[...]
    n1 = n1_ref[...]
    n2 = n2_ref[...]
    n3 = n3_ref[...]
    n4 = n4_ref[...]

    def mm(a, w_ref):
        return lax.dot_general(a, w_ref[...], (((1,), (1,)), ((), ())),
                               preferred_element_type=jnp.float32)

    for _ in range(NUM_STEPS):
        state = mm(state, a_s_ref) + mm(static_input, a_x_ref)
        residual = state
        sn = _rms(state, n1)
        attn_out = mm(mm(sn, wv_ref), wo_ref)
        state = _rms(residual + attn_out, n2)
        residual = state
        h = mm(_rms(state, n3), m1_ref)
        h = h * jax.nn.sigmoid(h)
        state = _rms(residual + mm(h, m2_ref), n4)
    out_ref[...] = state


def _recurrent(eo0, eo1, state0, a_s, a_x, n1, n2, n3, n4, Wv, Wo, mW1, mW2):
    TB = 256
    tok = pl.BlockSpec((TB, D), lambda b: (b, 0))
    full = lambda shape: pl.BlockSpec(shape, lambda b: (0,) * len(shape))
    return pl.pallas_call(
        _recur_body,
        grid=(N // TB,),
        in_specs=[
            tok, tok, tok,
            full((D, D)), full((D, D)),
            full((1, D)), full((1, D)), full((1, D)), full((1, D)),
            full((D, D)), full((D, D)),
            full((MLPH, D)), full((D, MLPH)),
        ],
        out_specs=tok,
        out_shape=jax.ShapeDtypeStruct((N, D), jnp.float32),
    )(eo0, eo1, state0, a_s, a_x, n1, n2, n3, n4, Wv, Wo, mW1, mW2)


def kernel(x, Wr, W1, W2, adapter_W, norm1_w, norm2_w, norm3_w, norm4_w,
           Wq, Wk, Wv, Wo, mlp_W1, mlp_W2):
    x_flat = x.reshape(N, D)
    wr_pad = jnp.zeros((128, D), jnp.float32).at[:E].set(Wr)

    logits_pad, i1, i2, w1, w2, lb = _router(x_flat, wr_pad)
    router_logits = logits_pad[:, :E]
    topi = jnp.concatenate([i1, i2], axis=1)
    topw = jnp.concatenate([w1, w2], axis=1)

    assigned = topi.reshape(-1)
    oh = (assigned[:, None] == jnp.arange(E)[None, :]).astype(jnp.int32)
    within = jnp.take_along_axis(jnp.cumsum(oh, axis=0) - oh,
                                 assigned[:, None], axis=1)[:, 0]
    counts = jnp.sum(oh, axis=0)
    padded = ((counts + BLK - 1) // BLK) * BLK
    bounds = jnp.cumsum(padded)
    poff = bounds - padded
    dest = (poff[assigned] + within).astype(jnp.int32)
    src_tok = jnp.zeros((P,), jnp.int32).at[dest].set(
        (jnp.arange(NK) // K).astype(jnp.int32))
    w_sorted = jnp.zeros((P, 1), jnp.float32).at[dest, 0].set(topw.reshape(-1))
    block_expert = jnp.minimum(
        jnp.searchsorted(bounds, jnp.arange(NB, dtype=jnp.int32) * BLK,
                         side="right"),
        E - 1).astype(jnp.int32)

    xg = _sc_gather_P(x_flat, src_tok)
    y = _ffn(block_expert, xg, W1, W2, w_sorted)
    d = dest.reshape(N, K)
    eo0 = _sc_gather_N(y, d[:, 0])
    eo1 = _sc_gather_N(y, d[:, 1])

    state0 = 0.02 * jax.random.normal(
        jax.random.fold_in(jax.random.key(0), 123), (N, D), dtype=jnp.float32)
    a_s = adapter_W[:, :D]
    a_x = adapter_W[:, D:]
    final = _recurrent(eo0, eo1, state0, a_s, a_x,
                       norm1_w.reshape(1, D), norm2_w.reshape(1, D),
                       norm3_w.reshape(1, D), norm4_w.reshape(1, D),
                       Wv, Wo, mlp_W1, mlp_W2)
    return final.reshape(B, S, D), lb[0, 0], router_logits


# speedup vs baseline: 1.0731x; 1.0731x over previous
import functools

import jax
import jax.numpy as jnp
from jax import lax
from jax.experimental import pallas as pl
from jax.experimental.pallas import tpu as pltpu
from jax.experimental.pallas import tpu_sc as plsc

B, S, D = 1, 2048, 768
E, K = 16, 2
DFF = 2 * D
MLPH = 4 * D
NUM_STEPS = 2
N = B * S
NK = N * K

BLK = 128
NB = NK // BLK + E
P = NB * BLK

SC_NC, SC_NS = 2, 16
NW = SC_NC * SC_NS

_EPS = 1e-6
_NEG = -1e30


def _router_body(x_ref, wr_ref, logits_ref, i1_ref, i2_ref, w1_ref, w2_ref,
                 lb_ref):
    x = x_ref[...]
    logits = lax.dot_general(x, wr_ref[...], (((1,), (1,)), ((), ())),
                             preferred_element_type=jnp.float32)
    logits_ref[...] = logits
    col = lax.broadcasted_iota(jnp.int32, logits.shape, 1)
    valid = col < E
    lm = jnp.where(valid, logits, _NEG)
    m1 = jnp.max(lm, axis=1, keepdims=True)
    i1 = jnp.min(jnp.where(lm == m1, col, 127), axis=1, keepdims=True)
    lm2 = jnp.where(col == i1, _NEG, lm)
    m2 = jnp.max(lm2, axis=1, keepdims=True)
    i2 = jnp.min(jnp.where(lm2 == m2, col, 127), axis=1, keepdims=True)
    ex = jnp.where(valid, jnp.exp(lm - m1), 0.0)
    ssum = jnp.sum(ex, axis=1, keepdims=True)
    probs = ex / ssum
    p1 = 1.0 / ssum
    p2 = jnp.exp(m2 - m1) / ssum
    tot = p1 + p2
    i1_ref[...] = i1
    i2_ref[...] = i2
    w1_ref[...] = p1 / tot
    w2_ref[...] = p2 / tot
    assign = (col == i1).astype(jnp.float32) + (col == i2).astype(jnp.float32)
    ca = jnp.sum(assign, axis=0)
    cp = jnp.sum(probs, axis=0)
    lb_ref[...] = ((float(E) / (K * N * N)) * jnp.sum(ca * cp)).reshape(1, 1)


def _router(x_flat, wr_pad):
    return pl.pallas_call(
        _router_body,
        out_shape=(
            jax.ShapeDtypeStruct((N, 128), jnp.float32),
            jax.ShapeDtypeStruct((N, 1), jnp.int32),
            jax.ShapeDtypeStruct((N, 1), jnp.int32),
            jax.ShapeDtypeStruct((N, 1), jnp.float32),
            jax.ShapeDtypeStruct((N, 1), jnp.float32),
            jax.ShapeDtypeStruct((1, 1), jnp.float32),
        ),
    )(x_flat, wr_pad)


@functools.lru_cache(maxsize=None)
def _make_sc_gather(n_out, d):
    rpw = n_out // NW
    ch = min(rpw, 64)
    n_ch = rpw // ch
    mesh = plsc.VectorSubcoreMesh(core_axis_name="c", subcore_axis_name="s",
                                  num_cores=SC_NC, num_subcores=SC_NS)

    @functools.partial(
        pl.kernel,
        out_type=jax.ShapeDtypeStruct((n_out, d), jnp.float32),
        mesh=mesh,
        scratch_types=[
            pltpu.VMEM((ch,), jnp.int32),
            pltpu.VMEM((ch,), jnp.int32),
            pltpu.VMEM((2, ch, d), jnp.float32),
            pltpu.SemaphoreType.DMA,
            pltpu.SemaphoreType.DMA,
            pltpu.SemaphoreType.DMA,
            pltpu.SemaphoreType.DMA,
            pltpu.SemaphoreType.DMA,
            pltpu.SemaphoreType.DMA,
        ],
    )
    def gather_k(table_hbm, idx_hbm, out_hbm, ix0, ix1, rows_v,
                 g0, g1, w0, w1, i0, i1):
        wid = lax.axis_index("s") * SC_NC + lax.axis_index("c")
        base = wid * rpw
        ibuf = (ix0, ix1)
        gsem = (g0, g1)
        wsem = (w0, w1)
        isem = (i0, i1)
        gathers = [None] * n_ch
        writes = [None] * n_ch
        iloads = [None] * n_ch
        iloads[0] = pltpu.async_copy(idx_hbm.at[pl.ds(base, ch)], ix0, i0)
        for c in range(n_ch):
            s = c % 2
            if c >= 2:
                writes[c - 2].wait()
            iloads[c].wait()
            gathers[c] = pltpu.async_copy(
                table_hbm.at[ibuf[s]], rows_v.at[s], gsem[s])
            if c >= 1:
                gathers[c - 1].wait()
                writes[c - 1] = pltpu.async_copy(
                    rows_v.at[(c - 1) % 2],
                    out_hbm.at[pl.ds(base + (c - 1) * ch, ch)],
                    wsem[(c - 1) % 2])
            if c + 1 < n_ch:
                iloads[c + 1] = pltpu.async_copy(
                    idx_hbm.at[pl.ds(base + (c + 1) * ch, ch)],
                    ibuf[(c + 1) % 2], isem[(c + 1) % 2])
        gathers[n_ch - 1].wait()
        writes[n_ch - 1] = pltpu.async_copy(
            rows_v.at[(n_ch - 1) % 2],
            out_hbm.at[pl.ds(base + (n_ch - 1) * ch, ch)],
            wsem[(n_ch - 1) % 2])
        if n_ch >= 2:
            writes[n_ch - 2].wait()
        writes[n_ch - 1].wait()

    return gather_k


def _sc_gather_P(table, idx):
    return _make_sc_gather(P, D)(table, idx)


def _sc_gather_N(table, idx):
    return _make_sc_gather(N, D)(table, idx)


def _ffn_body(be_ref, xg_ref, w1_ref, w2_ref, ws_ref, y_ref):
    xb = xg_ref[...].astype(jnp.bfloat16)
    h = lax.dot_general(xb, w1_ref[0], (((1,), (1,)), ((), ())),
                        preferred_element_type=jnp.float32)
    h = 0.5 * h * (1.0 + lax.erf(h * (2.0 ** -0.5)))
    y = lax.dot_general(h.astype(jnp.bfloat16), w2_ref[0],
                        (((1,), (1,)), ((), ())),
                        preferred_element_type=jnp.float32)
    y_ref[...] = y * ws_ref[...]


def _ffn(block_expert, xg, W1, W2, ws):
    grid_spec = pltpu.PrefetchScalarGridSpec(
        num_scalar_prefetch=1,
        grid=(NB,),
        in_specs=[
            pl.BlockSpec((BLK, D), lambda b, be: (b, 0)),
            pl.BlockSpec((1, DFF, D), lambda b, be: (be[b], 0, 0)),
            pl.BlockSpec((1, D, DFF), lambda b, be: (be[b], 0, 0)),
            pl.BlockSpec((BLK, 1), lambda b, be: (b, 0)),
        ],
        out_specs=pl.BlockSpec((BLK, D), lambda b, be: (b, 0)),
    )
    return pl.pallas_call(
        _ffn_body,
        grid_spec=grid_spec,
        out_shape=jax.ShapeDtypeStruct((P, D), jnp.float32),
    )(block_expert, xg, W1, W2, ws)


def _rms(v, w):
    return v * lax.rsqrt(jnp.mean(v * v, axis=-1, keepdims=True) + _EPS) * w


def _recur_body(eo0_ref, eo1_ref, st_ref, a_s_ref, a_x_ref, n1_ref, n2_ref,
                n3_ref, n4_ref, wv_ref, wo_ref, m1_ref, m2_ref, out_ref):
    static_input = eo0_ref[...] + eo1_ref[...]
    state = st_ref[...]
    n1 = n1_ref[...]
    n2 = n2_ref[...]
    n3 = n3_ref[...]
    n4 = n4_ref[...]

    def mm(a, w_ref):
        return lax.dot_general(a, w_ref[...], (((1,), (1,)), ((), ())),
                               preferred_element_type=jnp.float32)

    for _ in range(NUM_STEPS):
        state = mm(state, a_s_ref) + mm(static_input, a_x_ref)
        residual = state
        sn = _rms(state, n1)
        attn_out = mm(mm(sn, wv_ref), wo_ref)
        state = _rms(residual + attn_out, n2)
        residual = state
        h = mm(_rms(state, n3), m1_ref)
        h = h * jax.nn.sigmoid(h)
        state = _rms(residual + mm(h, m2_ref), n4)
    out_ref[...] = state


def _recurrent(eo0, eo1, state0, a_s, a_x, n1, n2, n3, n4, Wv, Wo, mW1, mW2):
    TB = 256
    tok = pl.BlockSpec((TB, D), lambda b: (b, 0))
    full = lambda shape: pl.BlockSpec(shape, lambda b: (0,) * len(shape))
    return pl.pallas_call(
        _recur_body,
        grid=(N // TB,),
        in_specs=[
            tok, tok, tok,
            full((D, D)), full((D, D)),
            full((1, D)), full((1, D)), full((1, D)), full((1, D)),
            full((D, D)), full((D, D)),
            full((MLPH, D)), full((D, MLPH)),
        ],
        out_specs=tok,
        out_shape=jax.ShapeDtypeStruct((N, D), jnp.float32),
    )(eo0, eo1, state0, a_s, a_x, n1, n2, n3, n4, Wv, Wo, mW1, mW2)


def kernel(x, Wr, W1, W2, adapter_W, norm1_w, norm2_w, norm3_w, norm4_w,
           Wq, Wk, Wv, Wo, mlp_W1, mlp_W2):
    x_flat = x.reshape(N, D)
    wr_pad = jnp.zeros((128, D), jnp.float32).at[:E].set(Wr)

    logits_pad, i1, i2, w1, w2, lb = _router(x_flat, wr_pad)
    router_logits = logits_pad[:, :E]
    topi = jnp.concatenate([i1, i2], axis=1)
    topw = jnp.concatenate([w1, w2], axis=1)

    assigned = topi.reshape(-1)
    oh = (assigned[:, None] == jnp.arange(E)[None, :]).astype(jnp.int32)
    within = jnp.take_along_axis(jnp.cumsum(oh, axis=0) - oh,
                                 assigned[:, None], axis=1)[:, 0]
    counts = jnp.sum(oh, axis=0)
    padded = ((counts + BLK - 1) // BLK) * BLK
    bounds = jnp.cumsum(padded)
    poff = bounds - padded
    dest = (poff[assigned] + within).astype(jnp.int32)
    src_tok = (jnp.arange(P, dtype=jnp.int32) % N).at[dest].set(
        (jnp.arange(NK) // K).astype(jnp.int32))
    w_sorted = jnp.zeros((P, 1), jnp.float32).at[dest, 0].set(topw.reshape(-1))
    block_expert = jnp.minimum(
        jnp.searchsorted(bounds, jnp.arange(NB, dtype=jnp.int32) * BLK,
                         side="right"),
        E - 1).astype(jnp.int32)

    xg = _sc_gather_P(x_flat, src_tok)
    y = _ffn(block_expert, xg, W1.astype(jnp.bfloat16),
             W2.astype(jnp.bfloat16), w_sorted)
    d = dest.reshape(N, K)
    eo0 = _sc_gather_N(y, d[:, 0])
    eo1 = _sc_gather_N(y, d[:, 1])

    state0 = 0.02 * jax.random.normal(
        jax.random.fold_in(jax.random.key(0), 123), (N, D), dtype=jnp.float32)
    a_s = adapter_W[:, :D]
    a_x = adapter_W[:, D:]
    final = _recurrent(eo0, eo1, state0, a_s, a_x,
                       norm1_w.reshape(1, D), norm2_w.reshape(1, D),
                       norm3_w.reshape(1, D), norm4_w.reshape(1, D),
                       Wv, Wo, mlp_W1, mlp_W2)
    return final.reshape(B, S, D), lb[0, 0], router_logits


# speedup vs baseline: 1.2844x; 1.1969x over previous
import functools

import jax
import jax.numpy as jnp
from jax import lax
from jax.experimental import pallas as pl
from jax.experimental.pallas import tpu as pltpu
from jax.experimental.pallas import tpu_sc as plsc

B, S, D = 1, 2048, 768
E, K = 16, 2
DFF = 2 * D
MLPH = 4 * D
NUM_STEPS = 2
N = B * S
NK = N * K

BLK = 128
NB = NK // BLK + E
P = NB * BLK

SC_NC, SC_NS = 2, 16
NW = SC_NC * SC_NS

_EPS = 1e-6
_NEG = -1e30


def _router_body(x_ref, wr_ref, logits_ref, i1_ref, i2_ref, w1_ref, w2_ref,
                 lb_ref):
    x = x_ref[...]
    logits = lax.dot_general(x, wr_ref[...], (((1,), (1,)), ((), ())),
                             preferred_element_type=jnp.float32)
    logits_ref[...] = logits
    col = lax.broadcasted_iota(jnp.int32, logits.shape, 1)
    valid = col < E
    lm = jnp.where(valid, logits, _NEG)
    m1 = jnp.max(lm, axis=1, keepdims=True)
    i1 = jnp.min(jnp.where(lm == m1, col, 127), axis=1, keepdims=True)
    lm2 = jnp.where(col == i1, _NEG, lm)
    m2 = jnp.max(lm2, axis=1, keepdims=True)
    i2 = jnp.min(jnp.where(lm2 == m2, col, 127), axis=1, keepdims=True)
    ex = jnp.where(valid, jnp.exp(lm - m1), 0.0)
    ssum = jnp.sum(ex, axis=1, keepdims=True)
    probs = ex / ssum
    p1 = 1.0 / ssum
    p2 = jnp.exp(m2 - m1) / ssum
    tot = p1 + p2
    i1_ref[...] = i1
    i2_ref[...] = i2
    w1_ref[...] = p1 / tot
    w2_ref[...] = p2 / tot
    assign = (col == i1).astype(jnp.float32) + (col == i2).astype(jnp.float32)
    ca = jnp.sum(assign, axis=0)
    cp = jnp.sum(probs, axis=0)
    lb_ref[...] = ((float(E) / (K * N * N)) * jnp.sum(ca * cp)).reshape(1, 1)


def _router(x_flat, wr_pad):
    return pl.pallas_call(
        _router_body,
        out_shape=(
            jax.ShapeDtypeStruct((N, 128), jnp.float32),
            jax.ShapeDtypeStruct((N, 1), jnp.int32),
            jax.ShapeDtypeStruct((N, 1), jnp.int32),
            jax.ShapeDtypeStruct((N, 1), jnp.float32),
            jax.ShapeDtypeStruct((N, 1), jnp.float32),
            jax.ShapeDtypeStruct((1, 1), jnp.float32),
        ),
    )(x_flat, wr_pad)


@functools.lru_cache(maxsize=None)
def _make_sc_gather(n_out, d):
    rpw = n_out // NW
    ch = min(rpw, 64)
    n_ch = rpw // ch
    mesh = plsc.VectorSubcoreMesh(core_axis_name="c", subcore_axis_name="s",
                                  num_cores=SC_NC, num_subcores=SC_NS)

    @functools.partial(
        pl.kernel,
        out_type=jax.ShapeDtypeStruct((n_out, d), jnp.float32),
        mesh=mesh,
        scratch_types=[
            pltpu.VMEM((ch,), jnp.int32),
            pltpu.VMEM((ch,), jnp.int32),
            pltpu.VMEM((2, ch, d), jnp.float32),
            pltpu.SemaphoreType.DMA,
            pltpu.SemaphoreType.DMA,
            pltpu.SemaphoreType.DMA,
            pltpu.SemaphoreType.DMA,
            pltpu.SemaphoreType.DMA,
            pltpu.SemaphoreType.DMA,
        ],
    )
    def gather_k(table_hbm, idx_hbm, out_hbm, ix0, ix1, rows_v,
                 g0, g1, w0, w1, i0, i1):
        wid = lax.axis_index("s") * SC_NC + lax.axis_index("c")
        base = wid * rpw
        ibuf = (ix0, ix1)
        gsem = (g0, g1)
        wsem = (w0, w1)
        isem = (i0, i1)
        gathers = [None] * n_ch
        writes = [None] * n_ch
        iloads = [None] * n_ch
        iloads[0] = pltpu.async_copy(idx_hbm.at[pl.ds(base, ch)], ix0, i0)
        for c in range(n_ch):
            s = c % 2
            if c >= 2:
                writes[c - 2].wait()
            iloads[c].wait()
            gathers[c] = pltpu.async_copy(
                table_hbm.at[ibuf[s]], rows_v.at[s], gsem[s])
            if c >= 1:
                gathers[c - 1].wait()
                writes[c - 1] = pltpu.async_copy(
                    rows_v.at[(c - 1) % 2],
                    out_hbm.at[pl.ds(base + (c - 1) * ch, ch)],
                    wsem[(c - 1) % 2])
            if c + 1 < n_ch:
                iloads[c + 1] = pltpu.async_copy(
                    idx_hbm.at[pl.ds(base + (c + 1) * ch, ch)],
                    ibuf[(c + 1) % 2], isem[(c + 1) % 2])
        gathers[n_ch - 1].wait()
        writes[n_ch - 1] = pltpu.async_copy(
            rows_v.at[(n_ch - 1) % 2],
            out_hbm.at[pl.ds(base + (n_ch - 1) * ch, ch)],
            wsem[(n_ch - 1) % 2])
        if n_ch >= 2:
            writes[n_ch - 2].wait()
        writes[n_ch - 1].wait()

    return gather_k


def _sc_gather_P(table, idx):
    return _make_sc_gather(P, D)(table, idx)


def _sc_gather_N(table, idx):
    return _make_sc_gather(N, D)(table, idx)


def _ffn_body(be_ref, xg_ref, w1_ref, w2_ref, y_ref):
    xb = xg_ref[...]
    h = lax.dot_general(xb, w1_ref[0], (((1,), (1,)), ((), ())),
                        preferred_element_type=jnp.float32)
    h = 0.5 * h * (1.0 + lax.erf(h * (2.0 ** -0.5)))
    y_ref[...] = lax.dot_general(h, w2_ref[0], (((1,), (1,)), ((), ())),
                                 preferred_element_type=jnp.float32)


def _ffn(block_expert, xg, W1, W2):
    grid_spec = pltpu.PrefetchScalarGridSpec(
        num_scalar_prefetch=1,
        grid=(NB,),
        in_specs=[
            pl.BlockSpec((BLK, D), lambda b, be: (b, 0)),
            pl.BlockSpec((1, DFF, D), lambda b, be: (be[b], 0, 0)),
            pl.BlockSpec((1, D, DFF), lambda b, be: (be[b], 0, 0)),
        ],
        out_specs=pl.BlockSpec((BLK, D), lambda b, be: (b, 0)),
    )
    return pl.pallas_call(
        _ffn_body,
        grid_spec=grid_spec,
        out_shape=jax.ShapeDtypeStruct((P, D), jnp.float32),
    )(block_expert, xg, W1, W2)


def _rms(v, w):
    return v * lax.rsqrt(jnp.mean(v * v, axis=-1, keepdims=True) + _EPS) * w


def _recur_body(eo0_ref, eo1_ref, cw0_ref, cw1_ref, st_ref, a_s_ref, a_x_ref,
                n1_ref, n2_ref, n3_ref, n4_ref, wv_ref, wo_ref, m1_ref,
                m2_ref, out_ref):
    static_input = (eo0_ref[...] * cw0_ref[...] + eo1_ref[...] * cw1_ref[...])
    state = st_ref[...]
    n1 = n1_ref[...]
    n2 = n2_ref[...]
    n3 = n3_ref[...]
    n4 = n4_ref[...]

    def mm(a, w_ref):
        return lax.dot_general(a, w_ref[...], (((1,), (1,)), ((), ())),
                               preferred_element_type=jnp.float32)

    for _ in range(NUM_STEPS):
        state = mm(state, a_s_ref) + mm(static_input, a_x_ref)
        residual = state
        sn = _rms(state, n1)
        attn_out = mm(mm(sn, wv_ref), wo_ref)
        state = _rms(residual + attn_out, n2)
        residual = state
        h = mm(_rms(state, n3), m1_ref)
        h = h * jax.nn.sigmoid(h)
        state = _rms(residual + mm(h, m2_ref), n4)
    out_ref[...] = state


def _recurrent(eo0, eo1, cw0, cw1, state0, a_s, a_x, n1, n2, n3, n4,
               Wv, Wo, mW1, mW2):
    TB = 256
    tok = pl.BlockSpec((TB, D), lambda b: (b, 0))
    wcol = pl.BlockSpec((TB, 1), lambda b: (b, 0))
    full = lambda shape: pl.BlockSpec(shape, lambda b: (0,) * len(shape))
    return pl.pallas_call(
        _recur_body,
        grid=(N // TB,),
        in_specs=[
            tok, tok, wcol, wcol, tok,
            full((D, D)), full((D, D)),
            full((1, D)), full((1, D)), full((1, D)), full((1, D)),
            full((D, D)), full((D, D)),
            full((MLPH, D)), full((D, MLPH)),
        ],
        out_specs=tok,
        out_shape=jax.ShapeDtypeStruct((N, D), jnp.float32),
    )(eo0, eo1, cw0, cw1, state0, a_s, a_x, n1, n2, n3, n4, Wv, Wo, mW1, mW2)


def kernel(x, Wr, W1, W2, adapter_W, norm1_w, norm2_w, norm3_w, norm4_w,
           Wq, Wk, Wv, Wo, mlp_W1, mlp_W2):
    x_flat = x.reshape(N, D)
    wr_pad = jnp.zeros((128, D), jnp.float32).at[:E].set(Wr)

    logits_pad, i1, i2, w1, w2, lb = _router(x_flat, wr_pad)
    router_logits = logits_pad[:, :E]
    topi = jnp.concatenate([i1, i2], axis=1)
    topw = jnp.concatenate([w1, w2], axis=1)

    assigned = topi.reshape(-1)
    oh = (assigned[:, None] == jnp.arange(E)[None, :]).astype(jnp.int32)
    within = jnp.take_along_axis(jnp.cumsum(oh, axis=0) - oh,
                                 assigned[:, None], axis=1)[:, 0]
    counts = jnp.sum(oh, axis=0)
    padded = ((counts + BLK - 1) // BLK) * BLK
    bounds = jnp.cumsum(padded)
    poff = bounds - padded
    dest = (poff[assigned] + within).astype(jnp.int32)
    src_tok = (jnp.arange(P, dtype=jnp.int32) % N).at[dest].set(
        (jnp.arange(NK) // K).astype(jnp.int32))
    block_expert = jnp.minimum(
        jnp.searchsorted(bounds, jnp.arange(NB, dtype=jnp.int32) * BLK,
                         side="right"),
        E - 1).astype(jnp.int32)

    xg = _sc_gather_P(x_flat, src_tok)
    y = _ffn(block_expert, xg, W1, W2)
    d = dest.reshape(N, K)
    eo0 = _sc_gather_N(y, d[:, 0])
    eo1 = _sc_gather_N(y, d[:, 1])

    state0 = 0.02 * jax.random.normal(
        jax.random.fold_in(jax.random.key(0), 123), (N, D), dtype=jnp.float32)
    a_s = adapter_W[:, :D]
    a_x = adapter_W[:, D:]
    final = _recurrent(eo0, eo1, w1, w2, state0, a_s, a_x,
                       norm1_w.reshape(1, D), norm2_w.reshape(1, D),
                       norm3_w.reshape(1, D), norm4_w.reshape(1, D),
                       Wv, Wo, mlp_W1, mlp_W2)
    return final.reshape(B, S, D), lb[0, 0], router_logits
